# baseline (speedup 1.0000x reference)
import functools

import jax
import jax.numpy as jnp
from jax import lax
from jax.experimental import pallas as pl
from jax.experimental.pallas import tpu as pltpu
from jax.experimental.pallas import tpu_sc as plsc

VT = 512


def _make_sc_gather(V, D, B):
    info = plsc.get_sparse_core_info()
    NC, NS = info.num_cores, info.num_subcores
    NW = NC * NS
    assert B % NW == 0 and (B // NW) % 8 == 0
    b_per_w = B // NW
    mesh = plsc.VectorSubcoreMesh(core_axis_name="c", subcore_axis_name="s")

    @functools.partial(
        pl.kernel, mesh=mesh,
        out_type=jax.ShapeDtypeStruct((B, D), jnp.float32),
        scratch_types=[
            pltpu.VMEM((b_per_w,), jnp.int32),
            pltpu.VMEM((b_per_w, D), jnp.float32),
            pltpu.SemaphoreType.DMA,
        ],
    )
    def sc_gather(table_hbm, idx_hbm, out_hbm, idx_v, rows_v, sem):
        wid = lax.axis_index("s") * NC + lax.axis_index("c")
        base = wid * b_per_w
        pltpu.sync_copy(idx_hbm.at[pl.ds(base, b_per_w)], idx_v)
        pltpu.async_copy(table_hbm.at[idx_v], rows_v, sem).wait()
        pltpu.sync_copy(rows_v, out_hbm.at[pl.ds(base, b_per_w)])

    return sc_gather


def _tc_head_body(tok_ref, pos_ref, w_ref, b_ref, tgt_ref,
                  out_ref, loss_ref, m_s, s_s, p_s, *, T, VOCAB, nV):
    vi = pl.program_id(0)

    @pl.when(vi == 0)
    def _init():
        m_s[...] = jnp.full((T, 128), -jnp.inf, dtype=jnp.float32)
        s_s[...] = jnp.zeros((T, 128), dtype=jnp.float32)
        p_s[...] = jnp.zeros((T, 128), dtype=jnp.float32)

    combined = tok_ref[...] + pos_ref[...]
    logits = jnp.dot(combined, w_ref[...],
                     preferred_element_type=jnp.float32) + b_ref[0, :][None, :]
    out_ref[...] = logits

    col = vi * VT + lax.broadcasted_iota(jnp.int32, (T, VT), 1)
    masked = jnp.where(col < VOCAB, logits, -jnp.inf)

    tile_max = jnp.max(masked, axis=1, keepdims=True)
    m_prev = m_s[:, 0:1]
    m_new = jnp.maximum(m_prev, tile_max)
    tile_sum = jnp.sum(jnp.exp(masked - m_new), axis=1, keepdims=True)
    s_new = s_s[:, 0:1] * jnp.exp(m_prev - m_new) + tile_sum
    m_s[...] = jnp.broadcast_to(m_new, (T, 128))
    s_s[...] = jnp.broadcast_to(s_new, (T, 128))

    tgt = tgt_ref[...]
    pick = jnp.sum(jnp.where(col == tgt, masked, 0.0), axis=1, keepdims=True)
    p_s[...] = p_s[:, 0:1] + pick + jnp.zeros((T, 128), dtype=jnp.float32)

    @pl.when(vi == nV - 1)
    def _fin():
        logz = m_s[:, 0:1] + jnp.log(s_s[:, 0:1])
        loss_ref[0, 0] = jnp.sum(logz - p_s[:, 0:1]) / T


def _tc_head(tok_rows, pos, W, b2d, tgt2d, T, D, VOCAB):
    nV = pl.cdiv(VOCAB, VT)
    body = functools.partial(_tc_head_body, T=T, VOCAB=VOCAB, nV=nV)
    return pl.pallas_call(
        body,
        grid=(nV,),
        in_specs=[
            pl.BlockSpec((T, D), lambda v: (0, 0)),
            pl.BlockSpec((T, D), lambda v: (0, 0)),
            pl.BlockSpec((D, VT), lambda v: (0, v)),
            pl.BlockSpec((1, VT), lambda v: (0, v)),
            pl.BlockSpec((T, 1), lambda v: (0, 0)),
        ],
        out_specs=[
            pl.BlockSpec((T, VT), lambda v: (0, v)),
            pl.BlockSpec(memory_space=pltpu.SMEM, block_shape=(1, 1),
                         index_map=lambda v: (0, 0)),
        ],
        out_shape=[
            jax.ShapeDtypeStruct((T, VOCAB), jnp.float32),
            jax.ShapeDtypeStruct((1, 1), jnp.float32),
        ],
        scratch_shapes=[
            pltpu.VMEM((T, 128), jnp.float32),
            pltpu.VMEM((T, 128), jnp.float32),
            pltpu.VMEM((T, 128), jnp.float32),
        ],
        compiler_params=pltpu.CompilerParams(
            dimension_semantics=("arbitrary",)),
    )(tok_rows, pos, W, b2d, tgt2d)


def kernel(index, targets, tok_emb, pos_emb, W, b):
    Bsz, T = index.shape
    V, D = tok_emb.shape
    VOCAB = W.shape[1]
    idx = index.reshape(Bsz * T)
    tok_rows = _make_sc_gather(V, D, Bsz * T)(tok_emb, idx)
    logits2d, loss11 = _tc_head(
        tok_rows, pos_emb[:T], W, b.reshape(1, VOCAB),
        targets.reshape(Bsz * T, 1), Bsz * T, D, VOCAB)
    return logits2d.reshape(Bsz, T, VOCAB), loss11.reshape(())
